# baseline (speedup 1.0000x reference)
import numpy as np
import jax
import jax.numpy as jnp
from jax.experimental import pallas as pl

BS, K, N = 16, 8, 128
H, W = 256, 256
OUT_H, OUT_W = 512, 256
C = BS * K


def _cv_kernel(tp_ref, vp_ref, mm_ref, out_ref):
    t0 = tp_ref[0]
    v0 = vp_ref[0]
    vmin = mm_ref[0, 0, 0]
    vmax = mm_ref[0, 0, 1]
    step_t = np.float32(1.0 / (H - 1))
    t = t0 / step_t
    step_v = (vmax - vmin) / np.float32(W - 1)
    v = (v0 - vmin) / step_v
    ts = jnp.where(t > 0.0, t, np.float32(1e9))

    T = jnp.broadcast_to(ts, (N, H))
    V = jnp.broadcast_to(v, (N, H))
    Q = jax.lax.broadcasted_iota(jnp.int32, (N, H), 1).astype(jnp.float32)
    idx = jax.lax.broadcasted_iota(jnp.int32, (N, H), 0)

    le = T <= Q
    cnt = jnp.sum(le.astype(jnp.int32), axis=0, keepdims=True)
    neg = np.float32(-3e38)
    pos = np.float32(3e38)
    t_lo = jnp.max(jnp.where(le, T, neg), axis=0, keepdims=True)
    i_lo = jnp.max(jnp.where(le & (T == t_lo), idx, -1), axis=0, keepdims=True)
    v_lo = jnp.sum(jnp.where(idx == i_lo, V, 0.0), axis=0, keepdims=True)
    gt = jnp.logical_not(le)
    t_hi = jnp.min(jnp.where(gt, T, pos), axis=0, keepdims=True)
    i_hi = jnp.min(jnp.where(gt & (T == t_hi), idx, N + 1), axis=0, keepdims=True)
    v_hi = jnp.sum(jnp.where(idx == i_hi, V, 0.0), axis=0, keepdims=True)

    q = jax.lax.broadcasted_iota(jnp.int32, (1, H), 1).astype(jnp.float32)
    interp = v_lo + (q - t_lo) / (t_hi - t_lo) * (v_hi - v_lo)
    vq = jnp.where(cnt == 0, v_hi, jnp.where(cnt == N, v_lo, interp))

    vqT = jnp.swapaxes(jnp.broadcast_to(vq, (8, H)), 0, 1)
    vq_col = vqT[:, 0:1]
    vi = jnp.clip(jnp.round(vq_col), 0, W - 1).astype(jnp.int32)

    vi_prev = jnp.concatenate([vi[0:1], vi[:-1]], axis=0)
    vi_next = jnp.concatenate([vi[1:], vi[-1:]], axis=0)

    cols = jax.lax.broadcasted_iota(jnp.int32, (H, W), 1)
    base = np.float32(0.01)
    hit = jnp.where(cols == vi, np.float32(0.675), np.float32(0.0))
    even = base + hit + jnp.where(cols == vi_prev, np.float32(0.225), np.float32(0.0))
    odd = base + hit + jnp.where(cols == vi_next, np.float32(0.225), np.float32(0.0))
    out_ref[0, :, 0, :] = even
    out_ref[0, :, 1, :] = odd


@jax.jit
def kernel(VelPoints, VMM):
    P = VelPoints.reshape(C, N, 2)
    tp = P[:, :, 0:1]
    vp = P[:, :, 1:2]
    mm = jnp.repeat(VMM, K, axis=0)[:, None, :]
    out = pl.pallas_call(
        _cv_kernel,
        grid=(C,),
        in_specs=[
            pl.BlockSpec((1, N, 1), lambda i: (i, 0, 0)),
            pl.BlockSpec((1, N, 1), lambda i: (i, 0, 0)),
            pl.BlockSpec((1, 1, 2), lambda i: (i, 0, 0)),
        ],
        out_specs=pl.BlockSpec((1, H, 2, W), lambda i: (i, 0, 0, 0)),
        out_shape=jax.ShapeDtypeStruct((C, H, 2, W), jnp.float32),
    )(tp, vp, mm)
    return out.reshape(BS, K, OUT_H, OUT_W)


# speedup vs baseline: 1.2071x; 1.2071x over previous
import numpy as np
import jax
import jax.numpy as jnp
from jax.experimental import pallas as pl

BS, K, N = 16, 8, 128
H, W = 256, 256
OUT_H, OUT_W = 512, 256
C = BS * K


CPB = 4


def _cv_kernel(tp_ref, vp_ref, mm_ref, out_ref):
    step = pl.program_id(0)
    for c in range(CPB):
        _one_curve(tp_ref, vp_ref, mm_ref, out_ref, step * CPB + c, c)


def _one_curve(tp_ref, vp_ref, mm_ref, out_ref, cur, slot):
    t0 = tp_ref[cur]
    v0 = vp_ref[cur]
    vmin = mm_ref[cur, 0, 0]
    vmax = mm_ref[cur, 0, 1]
    step_t = np.float32(1.0 / (H - 1))
    t = t0 / step_t
    step_v = (vmax - vmin) / np.float32(W - 1)
    v = (v0 - vmin) / step_v
    ts = jnp.where(t > 0.0, t, np.float32(1e9))

    T = jnp.broadcast_to(ts, (N, H))
    V = jnp.broadcast_to(v, (N, H))
    Q = jax.lax.broadcasted_iota(jnp.int32, (N, H), 1).astype(jnp.float32)
    idx = jax.lax.broadcasted_iota(jnp.int32, (N, H), 0)

    le = T <= Q
    cnt = jnp.sum(le.astype(jnp.int32), axis=0, keepdims=True)
    neg = np.float32(-3e38)
    pos = np.float32(3e38)
    t_lo = jnp.max(jnp.where(le, T, neg), axis=0, keepdims=True)
    i_lo = jnp.max(jnp.where(le & (T == t_lo), idx, -1), axis=0, keepdims=True)
    v_lo = jnp.sum(jnp.where(idx == i_lo, V, 0.0), axis=0, keepdims=True)
    gt = jnp.logical_not(le)
    t_hi = jnp.min(jnp.where(gt, T, pos), axis=0, keepdims=True)
    i_hi = jnp.min(jnp.where(gt & (T == t_hi), idx, N + 1), axis=0, keepdims=True)
    v_hi = jnp.sum(jnp.where(idx == i_hi, V, 0.0), axis=0, keepdims=True)

    q = jax.lax.broadcasted_iota(jnp.int32, (1, H), 1).astype(jnp.float32)
    interp = v_lo + (q - t_lo) / (t_hi - t_lo) * (v_hi - v_lo)
    vq = jnp.where(cnt == 0, v_hi, jnp.where(cnt == N, v_lo, interp))

    vqT = jnp.swapaxes(jnp.broadcast_to(vq, (8, H)), 0, 1)
    vq_col = vqT[:, 0:1]
    vi = jnp.clip(jnp.round(vq_col), 0, W - 1).astype(jnp.int32)

    vi_prev = jnp.concatenate([vi[0:1], vi[:-1]], axis=0)
    vi_next = jnp.concatenate([vi[1:], vi[-1:]], axis=0)

    cols = jax.lax.broadcasted_iota(jnp.int32, (H, W), 1)
    base = np.float32(0.01)
    hit = jnp.where(cols == vi, np.float32(0.675), np.float32(0.0))
    even = base + hit + jnp.where(cols == vi_prev, np.float32(0.225), np.float32(0.0))
    odd = base + hit + jnp.where(cols == vi_next, np.float32(0.225), np.float32(0.0))
    out_ref[slot, :, 0, :] = even
    out_ref[slot, :, 1, :] = odd


@jax.jit
def kernel(VelPoints, VMM):
    P = VelPoints.reshape(C, N, 2)
    tp = P[:, :, 0:1]
    vp = P[:, :, 1:2]
    mm = jnp.repeat(VMM, K, axis=0)[:, None, :]
    out = pl.pallas_call(
        _cv_kernel,
        grid=(C // CPB,),
        in_specs=[
            pl.BlockSpec((C, N, 1), lambda i: (0, 0, 0)),
            pl.BlockSpec((C, N, 1), lambda i: (0, 0, 0)),
            pl.BlockSpec((C, 1, 2), lambda i: (0, 0, 0)),
        ],
        out_specs=pl.BlockSpec((CPB, H, 2, W), lambda i: (i, 0, 0, 0)),
        out_shape=jax.ShapeDtypeStruct((C, H, 2, W), jnp.float32),
    )(tp, vp, mm)
    return out.reshape(BS, K, OUT_H, OUT_W)


# speedup vs baseline: 1.4573x; 1.2073x over previous
import numpy as np
import jax
import jax.numpy as jnp
from jax.experimental import pallas as pl

BS, K, N = 16, 8, 128
H, W = 256, 256
OUT_H, OUT_W = 512, 256
C = BS * K


CPB = 4


def _cv_kernel(tp_ref, vp_ref, mm_ref, out_ref):
    step = pl.program_id(0)
    for c in range(CPB):
        _one_curve(tp_ref, vp_ref, mm_ref, out_ref, step * CPB + c, c)


def _one_curve(tp_ref, vp_ref, mm_ref, out_ref, cur, slot):
    t0 = tp_ref[cur]
    v0 = vp_ref[cur]
    vmin = mm_ref[cur, 0, 0]
    vmax = mm_ref[cur, 0, 1]
    step_t = np.float32(1.0 / (H - 1))
    t = t0 / step_t
    step_v = (vmax - vmin) / np.float32(W - 1)
    v = (v0 - vmin) / step_v
    ts = jnp.where(t > 0.0, t, np.float32(1e9))

    T = jnp.broadcast_to(ts, (N, H))
    V = jnp.broadcast_to(v, (N, H))
    Q = jax.lax.broadcasted_iota(jnp.int32, (N, H), 1).astype(jnp.float32)
    idx = jax.lax.broadcasted_iota(jnp.int32, (N, H), 0)

    le = T <= Q
    cnt = jnp.sum(le.astype(jnp.int32), axis=0, keepdims=True)
    neg = np.float32(-3e38)
    pos = np.float32(3e38)
    t_lo = jnp.max(jnp.where(le, T, neg), axis=0, keepdims=True)
    i_lo = jnp.max(jnp.where(le & (T == t_lo), idx, -1), axis=0, keepdims=True)
    v_lo = jnp.sum(jnp.where(idx == i_lo, V, 0.0), axis=0, keepdims=True)
    gt = jnp.logical_not(le)
    t_hi = jnp.min(jnp.where(gt, T, pos), axis=0, keepdims=True)
    i_hi = jnp.min(jnp.where(gt & (T == t_hi), idx, N + 1), axis=0, keepdims=True)
    v_hi = jnp.sum(jnp.where(idx == i_hi, V, 0.0), axis=0, keepdims=True)

    q = jax.lax.broadcasted_iota(jnp.int32, (1, H), 1).astype(jnp.float32)
    interp = v_lo + (q - t_lo) / (t_hi - t_lo) * (v_hi - v_lo)
    vq = jnp.where(cnt == 0, v_hi, jnp.where(cnt == N, v_lo, interp))

    vqT = jnp.swapaxes(jnp.broadcast_to(vq, (8, H)), 0, 1)
    vq_col = vqT[:, 0:1]
    vi = jnp.clip(jnp.round(vq_col), 0, W - 1).astype(jnp.int32)

    vi_prev = jnp.concatenate([vi[0:1], vi[:-1]], axis=0)
    vi_next = jnp.concatenate([vi[1:], vi[-1:]], axis=0)

    cols = jax.lax.broadcasted_iota(jnp.int32, (H, W), 1)
    base = np.float32(0.01)
    hit = jnp.where(cols == vi, np.float32(0.675), np.float32(0.0))
    even = base + hit + jnp.where(cols == vi_prev, np.float32(0.225), np.float32(0.0))
    odd = base + hit + jnp.where(cols == vi_next, np.float32(0.225), np.float32(0.0))
    out_ref[slot] = jnp.concatenate([even, odd], axis=1)


@jax.jit
def kernel(VelPoints, VMM):
    P = VelPoints.reshape(C, N, 2)
    tp = P[:, :, 0:1]
    vp = P[:, :, 1:2]
    mm = jnp.repeat(VMM, K, axis=0)[:, None, :]
    out = pl.pallas_call(
        _cv_kernel,
        grid=(C // CPB,),
        in_specs=[
            pl.BlockSpec((C, N, 1), lambda i: (0, 0, 0)),
            pl.BlockSpec((C, N, 1), lambda i: (0, 0, 0)),
            pl.BlockSpec((C, 1, 2), lambda i: (0, 0, 0)),
        ],
        out_specs=pl.BlockSpec((CPB, H, 2 * W), lambda i: (i, 0, 0)),
        out_shape=jax.ShapeDtypeStruct((C, H, 2 * W), jnp.float32),
    )(tp, vp, mm)
    return out.reshape(BS, K, OUT_H, OUT_W)
